# baseline (speedup 1.0000x reference)
import functools

import jax
import jax.numpy as jnp
from jax import lax
from jax.experimental import pallas as pl
from jax.experimental.pallas import tpu as pltpu
from jax.experimental.pallas import tpu_sc as plsc

N_CORES = 2
N_SUB = 16
N_WORKERS = N_CORES * N_SUB
N_PIPE = 5


def _mm1_body(x_ref, w_ref, o_ref):
    o_ref[...] = lax.dot_general(
        w_ref[...], x_ref[...],
        dimension_numbers=(((0,), (1,)), ((), ())),
        preferred_element_type=jnp.float32)


def _act_body(p_ref, y_ref, b_ref, s_ref, o_ref):
    z = p_ref[0] + p_ref[1] + s_ref[0, 0] * y_ref[...] + b_ref[...]
    o_ref[...] = jax.nn.sigmoid(z)


def _mm2_body(q_ref, h_ref, w_ref, b_ref, s_ref, o_ref):
    d = w_ref.shape[0]
    z = q_ref[0] + q_ref[1] + s_ref[0, 0] * h_ref[...]
    for j in range(8):
        o_ref[:, j, :] = jnp.dot(z[:, j * d:(j + 1) * d], w_ref[...],
                                 preferred_element_type=jnp.float32) + b_ref[...]


@functools.cache
def _make_sc_agg(n_sc, d, e_tile):
    rows_per_tile = n_sc // N_SUB
    sub = e_tile // N_PIPE
    mesh = plsc.VectorSubcoreMesh(core_axis_name="c", subcore_axis_name="s")

    @functools.partial(
        pl.kernel,
        mesh=mesh,
        out_type=jax.ShapeDtypeStruct((N_CORES, n_sc, d), jnp.float32),
        scratch_types=[
            pltpu.VMEM((e_tile,), jnp.int32),
            pltpu.VMEM((e_tile,), jnp.int32),
            pltpu.VMEM((e_tile, d), jnp.float32),
            pltpu.VMEM_SHARED((n_sc, d), jnp.float32),
            [pltpu.SemaphoreType.DMA] * N_PIPE,
            pltpu.SemaphoreType.DMA,
        ],
        compiler_params=pltpu.CompilerParams(use_tc_tiling_on_sc=False),
    )
    def sc_agg(y_hbm, ei_hbm, zeros_hbm, out_hbm, src_v, dst_v, msgs, acc,
               sems_g, sem_s):
        c = lax.axis_index("c")
        s = lax.axis_index("s")
        wid = s * N_CORES + c
        base = wid * e_tile

        pltpu.sync_copy(ei_hbm.at[0, pl.ds(base, e_tile)], src_v)
        pltpu.sync_copy(ei_hbm.at[1, pl.ds(base, e_tile)], dst_v)
        gathers = [
            pltpu.async_copy(y_hbm.at[src_v.at[pl.ds(j * sub, sub)]],
                             msgs.at[pl.ds(j * sub, sub)], sems_g[j])
            for j in range(N_PIPE)
        ]
        pltpu.sync_copy(zeros_hbm.at[pl.ds(s * rows_per_tile, rows_per_tile)],
                        acc.at[pl.ds(s * rows_per_tile, rows_per_tile)])

        plsc.subcore_barrier()

        scatters = []
        for j in range(N_PIPE):
            gathers[j].wait()
            scatters.append(
                pltpu.async_copy(msgs.at[pl.ds(j * sub, sub)],
                                 acc.at[dst_v.at[pl.ds(j * sub, sub)]],
                                 sem_s, add=True))
        for sc_copy in scatters:
            sc_copy.wait()

        plsc.subcore_barrier()

        pltpu.sync_copy(acc.at[pl.ds(s * rows_per_tile, rows_per_tile)],
                        out_hbm.at[c, pl.ds(s * rows_per_tile, rows_per_tile)])

    return sc_agg


def kernel(x, edge_index, edge_weight, W1, b1, eps1, W2, b2, eps2):
    n, d_in = x.shape
    d_hid = W1.shape[1]
    d_out = W2.shape[1]
    n_edges = edge_index.shape[1]
    e_tile = n_edges // N_WORKERS
    d8 = 8 * d_hid
    gb = 128
    npk = -(-n // (8 * gb)) * 8 * gb
    rows = npk // 8
    ng = rows // gb

    ei = edge_index.astype(jnp.int32)
    zeros = jnp.zeros((npk, d_hid), jnp.float32)
    sc_agg = _make_sc_agg(npk, d_hid, e_tile)

    grid = (ng,)
    scale1 = (1.0 + eps1).astype(jnp.float32).reshape(1, 1)
    scale2 = (1.0 + eps2).astype(jnp.float32).reshape(1, 1)
    b1t = jnp.tile(b1.reshape(1, d_hid), (1, 8))

    mbx = 8 * gb
    y_t = pl.pallas_call(
        _mm1_body,
        grid=grid,
        in_specs=[pl.BlockSpec((mbx, d_in), lambda i: (i, 0)),
                  pl.BlockSpec((d_in, d_hid), lambda i: (0, 0))],
        out_specs=pl.BlockSpec((d_hid, mbx), lambda i: (0, i)),
        out_shape=jax.ShapeDtypeStruct((d_hid, npk), jnp.float32),
    )(x, W1)
    y = y_t.T
    y8 = y.reshape(rows, d8)

    p = sc_agg(y, ei, zeros)

    p8 = p.reshape(N_CORES, rows, d8)
    h8 = pl.pallas_call(
        _act_body,
        grid=grid,
        in_specs=[pl.BlockSpec((N_CORES, gb, d8), lambda i: (0, i, 0)),
                  pl.BlockSpec((gb, d8), lambda i: (i, 0)),
                  pl.BlockSpec((1, d8), lambda i: (0, 0)),
                  pl.BlockSpec((1, 1), lambda i: (0, 0))],
        out_specs=pl.BlockSpec((gb, d8), lambda i: (i, 0)),
        out_shape=jax.ShapeDtypeStruct((rows, d8), jnp.float32),
    )(p8, y8, b1t, scale1)

    q = sc_agg(h8.reshape(npk, d_hid), ei, zeros)

    q8 = q.reshape(N_CORES, rows, d8)
    out3 = pl.pallas_call(
        _mm2_body,
        grid=grid,
        in_specs=[pl.BlockSpec((N_CORES, gb, d8), lambda i: (0, i, 0)),
                  pl.BlockSpec((gb, d8), lambda i: (i, 0)),
                  pl.BlockSpec((d_hid, d_out), lambda i: (0, 0)),
                  pl.BlockSpec((1, d_out), lambda i: (0, 0)),
                  pl.BlockSpec((1, 1), lambda i: (0, 0))],
        out_specs=pl.BlockSpec((gb, 8, d_out), lambda i: (i, 0, 0)),
        out_shape=jax.ShapeDtypeStruct((n // 8, 8, d_out), jnp.float32),
    )(q8, h8, W2, b2.reshape(1, d_out), scale2)

    return out3.reshape(n, d_out)


# speedup vs baseline: 1.0699x; 1.0699x over previous
import functools

import jax
import jax.numpy as jnp
from jax import lax
from jax.experimental import pallas as pl
from jax.experimental.pallas import tpu as pltpu
from jax.experimental.pallas import tpu_sc as plsc

N_CORES = 2
N_SUB = 16
N_WORKERS = N_CORES * N_SUB
N_PIPE = 5


def _mm1_body(x_ref, w_ref, o_ref):
    o_ref[...] = jnp.concatenate(
        [jnp.dot(x_ref[:, j, :], w_ref[...],
                 preferred_element_type=jnp.float32) for j in range(8)],
        axis=1)


def _act_body(p_ref, y_ref, b_ref, s_ref, o_ref):
    z = p_ref[0] + p_ref[1] + s_ref[0, 0] * y_ref[...] + b_ref[...]
    o_ref[...] = jax.nn.sigmoid(z)


def _mm2_body(q_ref, h_ref, w_ref, b_ref, s_ref, o_ref):
    d = w_ref.shape[0]
    z = q_ref[0] + q_ref[1] + s_ref[0, 0] * h_ref[...]
    for j in range(8):
        o_ref[:, j, :] = jnp.dot(z[:, j * d:(j + 1) * d], w_ref[...],
                                 preferred_element_type=jnp.float32) + b_ref[...]


@functools.cache
def _make_sc_agg(n_sc, d, e_tile):
    rows_per_tile = n_sc // N_SUB
    sub = e_tile // N_PIPE
    mesh = plsc.VectorSubcoreMesh(core_axis_name="c", subcore_axis_name="s")

    @functools.partial(
        pl.kernel,
        mesh=mesh,
        out_type=jax.ShapeDtypeStruct((N_CORES, n_sc, d), jnp.float32),
        scratch_types=[
            pltpu.VMEM((e_tile,), jnp.int32),
            pltpu.VMEM((e_tile,), jnp.int32),
            pltpu.VMEM((e_tile, d), jnp.float32),
            pltpu.VMEM_SHARED((n_sc, d), jnp.float32),
            pltpu.VMEM_SHARED((n_sc, d), jnp.float32),
            [pltpu.SemaphoreType.DMA] * N_PIPE,
            pltpu.SemaphoreType.DMA,
        ],
        compiler_params=pltpu.CompilerParams(use_tc_tiling_on_sc=False),
    )
    def sc_agg(y_hbm, ei_hbm, zeros_hbm, out_hbm, src_v, dst_v, msgs, acc,
               y_spm, sems_g, sem_s):
        c = lax.axis_index("c")
        s = lax.axis_index("s")
        wid = s * N_CORES + c
        base = wid * e_tile

        pltpu.sync_copy(ei_hbm.at[0, pl.ds(base, e_tile)], src_v)
        pltpu.sync_copy(ei_hbm.at[1, pl.ds(base, e_tile)], dst_v)
        pltpu.sync_copy(y_hbm.at[pl.ds(s * rows_per_tile, rows_per_tile)],
                        y_spm.at[pl.ds(s * rows_per_tile, rows_per_tile)])
        pltpu.sync_copy(zeros_hbm.at[pl.ds(s * rows_per_tile, rows_per_tile)],
                        acc.at[pl.ds(s * rows_per_tile, rows_per_tile)])

        plsc.subcore_barrier()

        gathers = [
            pltpu.async_copy(y_spm.at[src_v.at[pl.ds(j * sub, sub)]],
                             msgs.at[pl.ds(j * sub, sub)], sems_g[j])
            for j in range(N_PIPE)
        ]

        scatters = []
        for j in range(N_PIPE):
            gathers[j].wait()
            scatters.append(
                pltpu.async_copy(msgs.at[pl.ds(j * sub, sub)],
                                 acc.at[dst_v.at[pl.ds(j * sub, sub)]],
                                 sem_s, add=True))
        for sc_copy in scatters:
            sc_copy.wait()

        plsc.subcore_barrier()

        pltpu.sync_copy(acc.at[pl.ds(s * rows_per_tile, rows_per_tile)],
                        out_hbm.at[c, pl.ds(s * rows_per_tile, rows_per_tile)])

    return sc_agg


def kernel(x, edge_index, edge_weight, W1, b1, eps1, W2, b2, eps2):
    n, d_in = x.shape
    d_hid = W1.shape[1]
    d_out = W2.shape[1]
    n_edges = edge_index.shape[1]
    e_tile = n_edges // N_WORKERS
    d8 = 8 * d_hid
    gb = 128
    npk = -(-n // (8 * gb)) * 8 * gb
    rows = npk // 8
    ng = rows // gb

    ei = edge_index.astype(jnp.int32)
    zeros = jnp.zeros((npk, d_hid), jnp.float32)
    sc_agg = _make_sc_agg(npk, d_hid, e_tile)

    grid = (ng,)
    scale1 = (1.0 + eps1).astype(jnp.float32).reshape(1, 1)
    scale2 = (1.0 + eps2).astype(jnp.float32).reshape(1, 1)
    b1t = jnp.tile(b1.reshape(1, d_hid), (1, 8))

    x3 = x.reshape(n // 8, 8, d_in)
    y8 = pl.pallas_call(
        _mm1_body,
        grid=grid,
        in_specs=[pl.BlockSpec((gb, 8, d_in), lambda i: (i, 0, 0)),
                  pl.BlockSpec((d_in, d_hid), lambda i: (0, 0))],
        out_specs=pl.BlockSpec((gb, d8), lambda i: (i, 0)),
        out_shape=jax.ShapeDtypeStruct((rows, d8), jnp.float32),
    )(x3, W1)

    p = sc_agg(y8.reshape(npk, d_hid), ei, zeros)

    p8 = p.reshape(N_CORES, rows, d8)
    h8 = pl.pallas_call(
        _act_body,
        grid=grid,
        in_specs=[pl.BlockSpec((N_CORES, gb, d8), lambda i: (0, i, 0)),
                  pl.BlockSpec((gb, d8), lambda i: (i, 0)),
                  pl.BlockSpec((1, d8), lambda i: (0, 0)),
                  pl.BlockSpec((1, 1), lambda i: (0, 0))],
        out_specs=pl.BlockSpec((gb, d8), lambda i: (i, 0)),
        out_shape=jax.ShapeDtypeStruct((rows, d8), jnp.float32),
    )(p8, y8, b1t, scale1)

    q = sc_agg(h8.reshape(npk, d_hid), ei, zeros)

    q8 = q.reshape(N_CORES, rows, d8)
    out3 = pl.pallas_call(
        _mm2_body,
        grid=grid,
        in_specs=[pl.BlockSpec((N_CORES, gb, d8), lambda i: (0, i, 0)),
                  pl.BlockSpec((gb, d8), lambda i: (i, 0)),
                  pl.BlockSpec((d_hid, d_out), lambda i: (0, 0)),
                  pl.BlockSpec((1, d_out), lambda i: (0, 0)),
                  pl.BlockSpec((1, 1), lambda i: (0, 0))],
        out_specs=pl.BlockSpec((gb, 8, d_out), lambda i: (i, 0, 0)),
        out_shape=jax.ShapeDtypeStruct((n // 8, 8, d_out), jnp.float32),
    )(q8, h8, W2, b2.reshape(1, d_out), scale2)

    return out3.reshape(n, d_out)


# speedup vs baseline: 1.1854x; 1.1080x over previous
import functools

import jax
import jax.numpy as jnp
from jax import lax
from jax.experimental import pallas as pl
from jax.experimental.pallas import tpu as pltpu
from jax.experimental.pallas import tpu_sc as plsc

N_CORES = 2
N_SUB = 16
N_WORKERS = N_CORES * N_SUB
N_PIPE = 5


def _mm1_body(x_ref, w_ref, o_ref):
    o_ref[...] = jnp.concatenate(
        [jnp.dot(x_ref[:, j, :], w_ref[...],
                 preferred_element_type=jnp.float32) for j in range(8)],
        axis=1)


def _act_body(p_ref, y_ref, b_ref, s_ref, o_ref):
    z = p_ref[0] + p_ref[1] + s_ref[0, 0] * y_ref[...] + b_ref[...]
    o_ref[...] = jax.nn.sigmoid(z)


def _mm2_body(q_ref, h_ref, w_ref, b_ref, s_ref, o_ref):
    d = w_ref.shape[0]
    z = q_ref[0] + q_ref[1] + s_ref[0, 0] * h_ref[...]
    for j in range(8):
        o_ref[:, j, :] = jnp.dot(z[:, j * d:(j + 1) * d], w_ref[...],
                                 preferred_element_type=jnp.float32) + b_ref[...]


@functools.cache
def _make_sc_agg(n_sc, d, e_tile):
    rows_per_tile = n_sc // N_SUB
    sub = e_tile // N_PIPE
    mesh = plsc.VectorSubcoreMesh(core_axis_name="c", subcore_axis_name="s")

    @functools.partial(
        pl.kernel,
        mesh=mesh,
        out_type=jax.ShapeDtypeStruct((N_CORES, n_sc, d), jnp.float32),
        scratch_types=[
            pltpu.VMEM((e_tile,), jnp.int32),
            pltpu.VMEM((e_tile,), jnp.int32),
            pltpu.VMEM((e_tile, d), jnp.float32),
            pltpu.VMEM_SHARED((n_sc, d), jnp.float32),
            pltpu.VMEM_SHARED((n_sc, d), jnp.float32),
            [pltpu.SemaphoreType.DMA] * N_PIPE,
            pltpu.SemaphoreType.DMA,
        ],
        compiler_params=pltpu.CompilerParams(use_tc_tiling_on_sc=False),
    )
    def sc_agg(y_hbm, ei_hbm, zeros_hbm, out_hbm, src_v, dst_v, msgs, acc,
               y_spm, sems_g, sem_s):
        c = lax.axis_index("c")
        s = lax.axis_index("s")
        wid = s * N_CORES + c
        base = wid * e_tile

        rslice = pl.ds(s * rows_per_tile, rows_per_tile)
        i1 = pltpu.async_copy(ei_hbm.at[0, pl.ds(base, e_tile)], src_v,
                              sems_g[0])
        i2 = pltpu.async_copy(ei_hbm.at[1, pl.ds(base, e_tile)], dst_v,
                              sems_g[1])
        i3 = pltpu.async_copy(y_hbm.at[rslice], y_spm.at[rslice], sem_s)
        i4 = pltpu.async_copy(zeros_hbm.at[rslice], acc.at[rslice], sem_s)
        i3.wait()
        i4.wait()
        plsc.subcore_barrier()
        i1.wait()
        i2.wait()

        gathers = [
            pltpu.async_copy(y_spm.at[src_v.at[pl.ds(j * sub, sub)]],
                             msgs.at[pl.ds(j * sub, sub)], sems_g[j])
            for j in range(N_PIPE)
        ]

        scatters = []
        for j in range(N_PIPE):
            gathers[j].wait()
            scatters.append(
                pltpu.async_copy(msgs.at[pl.ds(j * sub, sub)],
                                 acc.at[dst_v.at[pl.ds(j * sub, sub)]],
                                 sem_s, add=True))
        for sc_copy in scatters:
            sc_copy.wait()

        plsc.subcore_barrier()

        pltpu.sync_copy(acc.at[pl.ds(s * rows_per_tile, rows_per_tile)],
                        out_hbm.at[c, pl.ds(s * rows_per_tile, rows_per_tile)])

    return sc_agg


def kernel(x, edge_index, edge_weight, W1, b1, eps1, W2, b2, eps2):
    n, d_in = x.shape
    d_hid = W1.shape[1]
    d_out = W2.shape[1]
    n_edges = edge_index.shape[1]
    e_tile = n_edges // N_WORKERS
    d8 = 8 * d_hid
    gb = 128
    npk = -(-n // (8 * gb)) * 8 * gb
    rows = npk // 8
    ng = rows // gb

    ei = edge_index.astype(jnp.int32)
    zeros = jnp.zeros((npk, d_hid), jnp.float32)
    sc_agg = _make_sc_agg(npk, d_hid, e_tile)

    grid = (ng,)
    scale1 = (1.0 + eps1).astype(jnp.float32).reshape(1, 1)
    scale2 = (1.0 + eps2).astype(jnp.float32).reshape(1, 1)
    b1t = jnp.tile(b1.reshape(1, d_hid), (1, 8))

    x3 = x.reshape(n // 8, 8, d_in)
    y8 = pl.pallas_call(
        _mm1_body,
        grid=grid,
        in_specs=[pl.BlockSpec((gb, 8, d_in), lambda i: (i, 0, 0)),
                  pl.BlockSpec((d_in, d_hid), lambda i: (0, 0))],
        out_specs=pl.BlockSpec((gb, d8), lambda i: (i, 0)),
        out_shape=jax.ShapeDtypeStruct((rows, d8), jnp.float32),
    )(x3, W1)

    p = sc_agg(y8.reshape(npk, d_hid), ei, zeros)

    p8 = p.reshape(N_CORES, rows, d8)
    h8 = pl.pallas_call(
        _act_body,
        in_specs=[pl.BlockSpec((N_CORES, rows, d8), lambda: (0, 0, 0)),
                  pl.BlockSpec((rows, d8), lambda: (0, 0)),
                  pl.BlockSpec((1, d8), lambda: (0, 0)),
                  pl.BlockSpec((1, 1), lambda: (0, 0))],
        out_specs=pl.BlockSpec((rows, d8), lambda: (0, 0)),
        out_shape=jax.ShapeDtypeStruct((rows, d8), jnp.float32),
    )(p8, y8, b1t, scale1)

    q = sc_agg(h8.reshape(npk, d_hid), ei, zeros)

    q8 = q.reshape(N_CORES, rows, d8)
    out3 = pl.pallas_call(
        _mm2_body,
        grid=grid,
        in_specs=[pl.BlockSpec((N_CORES, gb, d8), lambda i: (0, i, 0)),
                  pl.BlockSpec((gb, d8), lambda i: (i, 0)),
                  pl.BlockSpec((d_hid, d_out), lambda i: (0, 0)),
                  pl.BlockSpec((1, d_out), lambda i: (0, 0)),
                  pl.BlockSpec((1, 1), lambda i: (0, 0))],
        out_specs=pl.BlockSpec((gb, 8, d_out), lambda i: (i, 0, 0)),
        out_shape=jax.ShapeDtypeStruct((n // 8, 8, d_out), jnp.float32),
    )(q8, h8, W2, b2.reshape(1, d_out), scale2)

    return out3.reshape(n, d_out)
